# baseline (speedup 1.0000x reference)
import functools

import jax
import jax.numpy as jnp
from jax import lax
from jax.experimental import pallas as pl
from jax.experimental.pallas import tpu as pltpu
from jax.experimental.pallas import tpu_sc as plsc

N = 10000
K = 16
IN_DIM = 256
OUT_DIM = 256
HALF = OUT_DIM // 2

NUM_WORKERS = 32
CHUNK_ROWS = 8
LANES = 16
NPAD = ((N + NUM_WORKERS * CHUNK_ROWS - 1) // (NUM_WORKERS * CHUNK_ROWS)
        ) * NUM_WORKERS * CHUNK_ROWS
ROWS_PER_WORKER = NPAD // NUM_WORKERS
TC_BLOCK = 2048


def _encode_top16(x):
    b = lax.bitcast_convert_type(x, jnp.int32)
    r = (b + jnp.int32(0x7FFF) + ((b >> 16) & jnp.int32(1))) & jnp.int32(-65536)
    return r ^ ((r >> 31) & jnp.int32(0x7FFF0000))


def _decode_top16(e):
    h = e ^ ((e >> 31) & jnp.int32(0x7FFF0000))
    return lax.bitcast_convert_type(h, jnp.float32)


def _tc_body(kx_ref, kp_ref, qx_ref, qp_ref, wa_ref, w3a_ref, wb_ref, w3b_ref,
             wi_ref, w3_ref, bias_ref, z_ref, c_ref):
    f32 = jnp.float32
    a = (jnp.dot(kx_ref[...], wa_ref[...], preferred_element_type=f32)
         + jnp.dot(kp_ref[...], w3a_ref[...], preferred_element_type=f32))
    b = (jnp.dot(kx_ref[...], wb_ref[...], preferred_element_type=f32)
         + jnp.dot(kp_ref[...], w3b_ref[...], preferred_element_type=f32))
    z_ref[...] = lax.shift_right_logical(_encode_top16(a), 16) | _encode_top16(b)
    c_ref[...] = (jnp.dot(qx_ref[...], wi_ref[...], preferred_element_type=f32)
                  - jnp.dot(qp_ref[...], w3_ref[...], preferred_element_type=f32)
                  + bias_ref[...])


def _tc_zc(kx, kp8, qx, qp8, wa, w3a, wb, w3b, wi, w3, bias2):
    grid = NPAD // TC_BLOCK
    full = lambda shape: pl.BlockSpec(shape, lambda i: (0,) * len(shape))
    return pl.pallas_call(
        _tc_body,
        grid=(grid,),
        in_specs=[
            pl.BlockSpec((TC_BLOCK, IN_DIM), lambda i: (i, 0)),
            pl.BlockSpec((TC_BLOCK, 8), lambda i: (i, 0)),
            pl.BlockSpec((TC_BLOCK, IN_DIM), lambda i: (i, 0)),
            pl.BlockSpec((TC_BLOCK, 8), lambda i: (i, 0)),
            full((IN_DIM, HALF)),
            full((8, HALF)),
            full((IN_DIM, HALF)),
            full((8, HALF)),
            full((IN_DIM, OUT_DIM)),
            full((8, OUT_DIM)),
            full((1, OUT_DIM)),
        ],
        out_specs=[
            pl.BlockSpec((TC_BLOCK, HALF), lambda i: (i, 0)),
            pl.BlockSpec((TC_BLOCK, OUT_DIM), lambda i: (i, 0)),
        ],
        out_shape=[
            jax.ShapeDtypeStruct((NPAD, HALF), jnp.int32),
            jax.ShapeDtypeStruct((NPAD, OUT_DIM), jnp.float32),
        ],
    )(kx, kp8, qx, qp8, wa, w3a, wb, w3b, wi, w3, bias2)


def _tc_epi_body(m_ref, c_ref, out_ref):
    m = m_ref[...]
    lo = _decode_top16(m << 16)
    hi = _decode_top16(m & jnp.int32(-65536))
    out_ref[...] = jnp.concatenate([lo, hi], axis=1) + c_ref[...]


def _tc_epilogue(m, c):
    grid = N // 2000
    return pl.pallas_call(
        _tc_epi_body,
        grid=(grid,),
        in_specs=[
            pl.BlockSpec((2000, HALF), lambda i: (i, 0)),
            pl.BlockSpec((2000, OUT_DIM), lambda i: (i, 0)),
        ],
        out_specs=pl.BlockSpec((2000, OUT_DIM), lambda i: (i, 0)),
        out_shape=jax.ShapeDtypeStruct((N, OUT_DIM), jnp.float32),
    )(m, c)


NCHUNKS = ROWS_PER_WORKER // CHUNK_ROWS


def _sc_body(z_hbm, idx_hbm, out_hbm, ib0, ib1, g0, g1, ob, sem0, sem1):
    wid = lax.axis_index("c") * 16 + lax.axis_index("s")
    row0 = wid * ROWS_PER_WORKER
    gather_rows = CHUNK_ROWS * K

    def fire(t, ib, gb, sem):
        fbase = (row0 + t * CHUNK_ROWS) * K
        pltpu.sync_copy(idx_hbm.at[pl.ds(fbase, gather_rows)], ib)
        pltpu.make_async_copy(z_hbm.at[ib], gb, sem).start()

    def wait_compute(t, ib, gb, sem):
        pltpu.make_async_copy(z_hbm.at[ib], gb, sem).wait()

        obase = t * CHUNK_ROWS
        for b in range(HALF // LANES):
            sl = pl.ds(b * LANES, LANES)
            for g in range(CHUNK_ROWS):
                base = g * K
                v = gb[base, sl]
                acc_lo = v << 16
                acc_hi = v
                for k in range(1, K):
                    v = gb[base + k, sl]
                    acc_lo = jnp.maximum(acc_lo, v << 16)
                    acc_hi = jnp.maximum(acc_hi, v)
                ob[obase + g, sl] = (lax.shift_right_logical(acc_lo, 16)
                                     | (acc_hi & jnp.int32(-65536)))

    fire(0, ib0, g0, sem0)

    def outer(j, carry):
        t0 = 2 * j
        fire(t0 + 1, ib1, g1, sem1)
        wait_compute(t0, ib0, g0, sem0)

        @pl.when(t0 + 2 < NCHUNKS)
        def _():
            fire(t0 + 2, ib0, g0, sem0)

        wait_compute(t0 + 1, ib1, g1, sem1)
        return carry

    lax.fori_loop(0, NCHUNKS // 2, outer, 0)
    pltpu.sync_copy(ob, out_hbm.at[pl.ds(row0, ROWS_PER_WORKER)])


@functools.cache
def _sc_call():
    return pl.kernel(
        _sc_body,
        out_type=jax.ShapeDtypeStruct((NPAD, HALF), jnp.int32),
        mesh=plsc.VectorSubcoreMesh(core_axis_name="c", subcore_axis_name="s"),
        scratch_types=[
            pltpu.VMEM((CHUNK_ROWS * K,), jnp.int32),
            pltpu.VMEM((CHUNK_ROWS * K,), jnp.int32),
            pltpu.VMEM((CHUNK_ROWS * K, HALF), jnp.int32),
            pltpu.VMEM((CHUNK_ROWS * K, HALF), jnp.int32),
            pltpu.VMEM((ROWS_PER_WORKER, HALF), jnp.int32),
            pltpu.SemaphoreType.DMA,
            pltpu.SemaphoreType.DMA,
        ],
    )


def kernel(query_pos, key_pos, idx_neighbors, query_x, key_x,
           W_xi, b_xi, W_xn, b_xn):
    kp8 = jnp.pad(key_pos, ((0, 0), (0, 5)))
    qp8 = jnp.pad(query_pos, ((0, 0), (0, 5)))
    w3 = jnp.pad(W_xn[:3], ((0, 5), (0, 0)))
    wx = W_xn[3:]
    wa, wb = wx[:, :HALF], wx[:, HALF:]
    w3a, w3b = w3[:, :HALF], w3[:, HALF:]
    bias2 = (b_xi + b_xn)[None, :]

    z, c = _tc_zc(key_x, kp8, query_x, qp8, wa, w3a, wb, w3b, W_xi, w3, bias2)

    idx_flat = jnp.pad(idx_neighbors.astype(jnp.int32).reshape(-1),
                       (0, (NPAD - N) * K))
    m = _sc_call()(z, idx_flat)
    return _tc_epilogue(m, c)


# speedup vs baseline: 1.0116x; 1.0116x over previous
import functools

import jax
import jax.numpy as jnp
from jax import lax
from jax.experimental import pallas as pl
from jax.experimental.pallas import tpu as pltpu
from jax.experimental.pallas import tpu_sc as plsc

N = 10000
K = 16
IN_DIM = 256
OUT_DIM = 256
HALF = OUT_DIM // 2

NUM_WORKERS = 32
CHUNK_ROWS = 8
LANES = 16
NPAD = ((N + NUM_WORKERS * CHUNK_ROWS - 1) // (NUM_WORKERS * CHUNK_ROWS)
        ) * NUM_WORKERS * CHUNK_ROWS
ROWS_PER_WORKER = NPAD // NUM_WORKERS
TC_BLOCK = 2048


def _encode_top16(x):
    b = lax.bitcast_convert_type(x, jnp.int32)
    r = (b + jnp.int32(0x7FFF) + ((b >> 16) & jnp.int32(1))) & jnp.int32(-65536)
    return r ^ ((r >> 31) & jnp.int32(0x7FFF0000))


def _decode_top16(e):
    h = e ^ ((e >> 31) & jnp.int32(0x7FFF0000))
    return lax.bitcast_convert_type(h, jnp.float32)


def _tc_body(kx_ref, kp_ref, qx_ref, qp_ref, wa_ref, w3a_ref, wb_ref, w3b_ref,
             wi_ref, w3_ref, bias_ref, z_ref, c_ref):
    f32 = jnp.float32
    a = (jnp.dot(kx_ref[...], wa_ref[...], preferred_element_type=f32)
         + jnp.dot(kp_ref[...], w3a_ref[...], preferred_element_type=f32))
    b = (jnp.dot(kx_ref[...], wb_ref[...], preferred_element_type=f32)
         + jnp.dot(kp_ref[...], w3b_ref[...], preferred_element_type=f32))
    z_ref[...] = lax.shift_right_logical(_encode_top16(a), 16) | _encode_top16(b)
    c_ref[...] = (jnp.dot(qx_ref[...], wi_ref[...], preferred_element_type=f32)
                  - jnp.dot(qp_ref[...], w3_ref[...], preferred_element_type=f32)
                  + bias_ref[...])


def _tc_zc(kx, kp8, qx, qp8, wa, w3a, wb, w3b, wi, w3, bias2):
    grid = NPAD // TC_BLOCK
    full = lambda shape: pl.BlockSpec(shape, lambda i: (0,) * len(shape))
    return pl.pallas_call(
        _tc_body,
        grid=(grid,),
        in_specs=[
            pl.BlockSpec((TC_BLOCK, IN_DIM), lambda i: (i, 0)),
            pl.BlockSpec((TC_BLOCK, 8), lambda i: (i, 0)),
            pl.BlockSpec((TC_BLOCK, IN_DIM), lambda i: (i, 0)),
            pl.BlockSpec((TC_BLOCK, 8), lambda i: (i, 0)),
            full((IN_DIM, HALF)),
            full((8, HALF)),
            full((IN_DIM, HALF)),
            full((8, HALF)),
            full((IN_DIM, OUT_DIM)),
            full((8, OUT_DIM)),
            full((1, OUT_DIM)),
        ],
        out_specs=[
            pl.BlockSpec((TC_BLOCK, HALF), lambda i: (i, 0)),
            pl.BlockSpec((TC_BLOCK, OUT_DIM), lambda i: (i, 0)),
        ],
        out_shape=[
            jax.ShapeDtypeStruct((NPAD, HALF), jnp.int32),
            jax.ShapeDtypeStruct((NPAD, OUT_DIM), jnp.float32),
        ],
    )(kx, kp8, qx, qp8, wa, w3a, wb, w3b, wi, w3, bias2)


def _tc_epi_body(m_ref, c_ref, out_ref):
    m = m_ref[...]
    lo = _decode_top16(m << 16)
    hi = _decode_top16(m & jnp.int32(-65536))
    out_ref[...] = jnp.concatenate([lo, hi], axis=1) + c_ref[...]


def _tc_epilogue(m, c):
    grid = N // 2000
    return pl.pallas_call(
        _tc_epi_body,
        grid=(grid,),
        in_specs=[
            pl.BlockSpec((2000, HALF), lambda i: (i, 0)),
            pl.BlockSpec((2000, OUT_DIM), lambda i: (i, 0)),
        ],
        out_specs=pl.BlockSpec((2000, OUT_DIM), lambda i: (i, 0)),
        out_shape=jax.ShapeDtypeStruct((N, OUT_DIM), jnp.float32),
    )(m, c)


NCHUNKS = ROWS_PER_WORKER // CHUNK_ROWS


def _sc_body(z_hbm, idx_hbm, out_hbm, ib0, ib1, g0, g1, ob, sem0, sem1):
    wid = lax.axis_index("c") * 16 + lax.axis_index("s")
    row0 = wid * ROWS_PER_WORKER
    gather_rows = CHUNK_ROWS * K

    def fire(t, ib, gb, sem):
        fbase = (row0 + t * CHUNK_ROWS) * K
        pltpu.sync_copy(idx_hbm.at[pl.ds(fbase, gather_rows)], ib)
        pltpu.make_async_copy(z_hbm.at[ib], gb, sem).start()

    def wait_compute(t, ib, gb, sem):
        pltpu.make_async_copy(z_hbm.at[ib], gb, sem).wait()

        for b in range(HALF // LANES):
            sl = pl.ds(b * LANES, LANES)
            for g in range(CHUNK_ROWS):
                base = g * K
                v = gb[base, sl]
                acc_lo = v << 16
                acc_hi = v
                for k in range(1, K):
                    v = gb[base + k, sl]
                    acc_lo = jnp.maximum(acc_lo, v << 16)
                    acc_hi = jnp.maximum(acc_hi, v)
                ob[g, sl] = (lax.shift_right_logical(acc_lo, 16)
                             | (acc_hi & jnp.int32(-65536)))

        pltpu.sync_copy(ob, out_hbm.at[pl.ds(row0 + t * CHUNK_ROWS,
                                             CHUNK_ROWS)])

    fire(0, ib0, g0, sem0)

    def outer(j, carry):
        t0 = 2 * j
        fire(t0 + 1, ib1, g1, sem1)
        wait_compute(t0, ib0, g0, sem0)

        @pl.when(t0 + 2 < NCHUNKS)
        def _():
            fire(t0 + 2, ib0, g0, sem0)

        wait_compute(t0 + 1, ib1, g1, sem1)
        return carry

    lax.fori_loop(0, NCHUNKS // 2, outer, 0)


@functools.cache
def _sc_call():
    return pl.kernel(
        _sc_body,
        out_type=jax.ShapeDtypeStruct((NPAD, HALF), jnp.int32),
        mesh=plsc.VectorSubcoreMesh(core_axis_name="c", subcore_axis_name="s"),
        scratch_types=[
            pltpu.VMEM((CHUNK_ROWS * K,), jnp.int32),
            pltpu.VMEM((CHUNK_ROWS * K,), jnp.int32),
            pltpu.VMEM((CHUNK_ROWS * K, HALF), jnp.int32),
            pltpu.VMEM((CHUNK_ROWS * K, HALF), jnp.int32),
            pltpu.VMEM((CHUNK_ROWS, HALF), jnp.int32),
            pltpu.SemaphoreType.DMA,
            pltpu.SemaphoreType.DMA,
        ],
    )


def kernel(query_pos, key_pos, idx_neighbors, query_x, key_x,
           W_xi, b_xi, W_xn, b_xn):
    kp8 = jnp.pad(key_pos, ((0, 0), (0, 5)))
    qp8 = jnp.pad(query_pos, ((0, 0), (0, 5)))
    w3 = jnp.pad(W_xn[:3], ((0, 5), (0, 0)))
    wx = W_xn[3:]
    wa, wb = wx[:, :HALF], wx[:, HALF:]
    w3a, w3b = w3[:, :HALF], w3[:, HALF:]
    bias2 = (b_xi + b_xn)[None, :]

    z, c = _tc_zc(key_x, kp8, query_x, qp8, wa, w3a, wb, w3b, W_xi, w3, bias2)

    idx_flat = jnp.pad(idx_neighbors.astype(jnp.int32).reshape(-1),
                       (0, (NPAD - N) * K))
    m = _sc_call()(z, idx_flat)
    return _tc_epilogue(m, c)


# speedup vs baseline: 1.4142x; 1.3980x over previous
import functools

import jax
import jax.numpy as jnp
from jax import lax
from jax.experimental import pallas as pl
from jax.experimental.pallas import tpu as pltpu
from jax.experimental.pallas import tpu_sc as plsc

N = 10000
K = 16
IN_DIM = 256
OUT_DIM = 256
HALF = OUT_DIM // 2

NUM_WORKERS = 32
CHUNK_ROWS = 8
LANES = 16
NPAD = ((N + NUM_WORKERS * CHUNK_ROWS - 1) // (NUM_WORKERS * CHUNK_ROWS)
        ) * NUM_WORKERS * CHUNK_ROWS
ROWS_PER_WORKER = NPAD // NUM_WORKERS
TC_BLOCK = 2048


def _encode_top16(x):
    b = lax.bitcast_convert_type(x, jnp.int32)
    r = (b + jnp.int32(0x7FFF) + ((b >> 16) & jnp.int32(1))) & jnp.int32(-65536)
    return r ^ ((r >> 31) & jnp.int32(0x7FFF0000))


def _decode_top16(e):
    h = e ^ ((e >> 31) & jnp.int32(0x7FFF0000))
    return lax.bitcast_convert_type(h, jnp.float32)


def _tc_body(kx_ref, kp_ref, qx_ref, qp_ref, wa_ref, w3a_ref, wb_ref, w3b_ref,
             wi_ref, w3_ref, bias_ref, z_ref, c_ref):
    f32 = jnp.float32
    a = (jnp.dot(kx_ref[...], wa_ref[...], preferred_element_type=f32)
         + jnp.dot(kp_ref[...], w3a_ref[...], preferred_element_type=f32))
    b = (jnp.dot(kx_ref[...], wb_ref[...], preferred_element_type=f32)
         + jnp.dot(kp_ref[...], w3b_ref[...], preferred_element_type=f32))
    z_ref[...] = lax.shift_right_logical(_encode_top16(a), 16) | _encode_top16(b)
    c_ref[...] = (jnp.dot(qx_ref[...], wi_ref[...], preferred_element_type=f32)
                  - jnp.dot(qp_ref[...], w3_ref[...], preferred_element_type=f32)
                  + bias_ref[...])


def _tc_zc(kx, kp8, qx, qp8, wa, w3a, wb, w3b, wi, w3, bias2):
    grid = NPAD // TC_BLOCK
    full = lambda shape: pl.BlockSpec(shape, lambda i: (0,) * len(shape))
    return pl.pallas_call(
        _tc_body,
        grid=(grid,),
        in_specs=[
            pl.BlockSpec((TC_BLOCK, IN_DIM), lambda i: (i, 0)),
            pl.BlockSpec((TC_BLOCK, 8), lambda i: (i, 0)),
            pl.BlockSpec((TC_BLOCK, IN_DIM), lambda i: (i, 0)),
            pl.BlockSpec((TC_BLOCK, 8), lambda i: (i, 0)),
            full((IN_DIM, HALF)),
            full((8, HALF)),
            full((IN_DIM, HALF)),
            full((8, HALF)),
            full((IN_DIM, OUT_DIM)),
            full((8, OUT_DIM)),
            full((1, OUT_DIM)),
        ],
        out_specs=[
            pl.BlockSpec((TC_BLOCK, HALF), lambda i: (i, 0)),
            pl.BlockSpec((TC_BLOCK, OUT_DIM), lambda i: (i, 0)),
        ],
        out_shape=[
            jax.ShapeDtypeStruct((NPAD, HALF), jnp.int32),
            jax.ShapeDtypeStruct((NPAD, OUT_DIM), jnp.float32),
        ],
    )(kx, kp8, qx, qp8, wa, w3a, wb, w3b, wi, w3, bias2)


def _tc_epi_body(m_ref, c_ref, out_ref):
    m = m_ref[...]
    lo = _decode_top16(m << 16)
    hi = _decode_top16(m & jnp.int32(-65536))
    out_ref[...] = jnp.concatenate([lo, hi], axis=1) + c_ref[...]


def _tc_epilogue(m, c):
    grid = N // 2000
    return pl.pallas_call(
        _tc_epi_body,
        grid=(grid,),
        in_specs=[
            pl.BlockSpec((2000, HALF), lambda i: (i, 0)),
            pl.BlockSpec((2000, OUT_DIM), lambda i: (i, 0)),
        ],
        out_specs=pl.BlockSpec((2000, OUT_DIM), lambda i: (i, 0)),
        out_shape=jax.ShapeDtypeStruct((N, OUT_DIM), jnp.float32),
    )(m, c)


NCHUNKS = ROWS_PER_WORKER // CHUNK_ROWS


def _sc_body(z_hbm, idx_hbm, out_hbm, ib0, ib1, g0, g1, ob, sem0, sem1):
    wid = lax.axis_index("c") * 16 + lax.axis_index("s")
    row0 = wid * ROWS_PER_WORKER
    gather_rows = CHUNK_ROWS * K

    nc = jnp.minimum(ROWS_PER_WORKER, N - row0) // CHUNK_ROWS

    def fire(t, ib, gb, sem):
        fbase = (row0 + t * CHUNK_ROWS) * K
        pltpu.sync_copy(idx_hbm.at[pl.ds(fbase, gather_rows)], ib)
        pltpu.make_async_copy(z_hbm.at[ib], gb, sem).start()

    def wait_compute(t, ib, gb, sem):
        pltpu.make_async_copy(z_hbm.at[ib], gb, sem).wait()

        for b in range(HALF // LANES):
            sl = pl.ds(b * LANES, LANES)
            for g in range(CHUNK_ROWS):
                base = g * K
                v = gb[base, sl]
                acc_lo = v << 16
                acc_hi = v
                for k in range(1, K):
                    v = gb[base + k, sl]
                    acc_lo = jnp.maximum(acc_lo, v << 16)
                    acc_hi = jnp.maximum(acc_hi, v)
                ob[g, sl] = (lax.shift_right_logical(acc_lo, 16)
                             | (acc_hi & jnp.int32(-65536)))

        pltpu.sync_copy(ob, out_hbm.at[pl.ds(row0 + t * CHUNK_ROWS,
                                             CHUNK_ROWS)])

    fire(0, ib0, g0, sem0)

    def outer(j, carry):
        t0 = 2 * j
        fire(t0 + 1, ib1, g1, sem1)
        wait_compute(t0, ib0, g0, sem0)

        @pl.when(t0 + 2 < nc)
        def _():
            fire(t0 + 2, ib0, g0, sem0)

        wait_compute(t0 + 1, ib1, g1, sem1)
        return carry

    lax.fori_loop(0, nc // 2, outer, 0)


@functools.cache
def _sc_call():
    return pl.kernel(
        _sc_body,
        out_type=jax.ShapeDtypeStruct((N, HALF), jnp.int32),
        mesh=plsc.VectorSubcoreMesh(core_axis_name="c", subcore_axis_name="s"),
        scratch_types=[
            pltpu.VMEM((CHUNK_ROWS * K,), jnp.int32),
            pltpu.VMEM((CHUNK_ROWS * K,), jnp.int32),
            pltpu.VMEM((CHUNK_ROWS * K, HALF), jnp.int32),
            pltpu.VMEM((CHUNK_ROWS * K, HALF), jnp.int32),
            pltpu.VMEM((CHUNK_ROWS, HALF), jnp.int32),
            pltpu.SemaphoreType.DMA,
            pltpu.SemaphoreType.DMA,
        ],
    )


def kernel(query_pos, key_pos, idx_neighbors, query_x, key_x,
           W_xi, b_xi, W_xn, b_xn):
    kp8 = jnp.pad(key_pos, ((0, 0), (0, 5)))
    qp8 = jnp.pad(query_pos, ((0, 0), (0, 5)))
    w3 = jnp.pad(W_xn[:3], ((0, 5), (0, 0)))
    wx = W_xn[3:]
    wa, wb = wx[:, :HALF], wx[:, HALF:]
    w3a, w3b = w3[:, :HALF], w3[:, HALF:]
    bias2 = (b_xi + b_xn)[None, :]

    z, c = _tc_zc(key_x, kp8, query_x, qp8, wa, w3a, wb, w3b, W_xi, w3, bias2)

    idx_flat = idx_neighbors.astype(jnp.int32).reshape(-1)
    m = _sc_call()(z, idx_flat)
    return _tc_epilogue(m, c)


# speedup vs baseline: 2.0102x; 1.4214x over previous
import functools

import jax
import jax.numpy as jnp
from jax import lax
from jax.experimental import pallas as pl
from jax.experimental.pallas import tpu as pltpu
from jax.experimental.pallas import tpu_sc as plsc

N = 10000
K = 16
IN_DIM = 256
OUT_DIM = 256
HALF = OUT_DIM // 2

NUM_WORKERS = 32
CHUNK_ROWS = 16
LANES = 16
NPAD = ((N + NUM_WORKERS * CHUNK_ROWS - 1) // (NUM_WORKERS * CHUNK_ROWS)
        ) * NUM_WORKERS * CHUNK_ROWS
ROWS_PER_WORKER = NPAD // NUM_WORKERS
TC_BLOCK = 2048


def _encode_top16(x):
    b = lax.bitcast_convert_type(x, jnp.int32)
    r = (b + jnp.int32(0x7FFF) + ((b >> 16) & jnp.int32(1))) & jnp.int32(-65536)
    return r ^ ((r >> 31) & jnp.int32(0x7FFF0000))


def _decode_top16(e):
    h = e ^ ((e >> 31) & jnp.int32(0x7FFF0000))
    return lax.bitcast_convert_type(h, jnp.float32)


def _tc_body(kx_ref, kp_ref, qx_ref, qp_ref, wa_ref, w3a_ref, wb_ref, w3b_ref,
             wi_ref, w3_ref, bias_ref, z_ref, c_ref):
    f32 = jnp.float32
    a = (jnp.dot(kx_ref[...], wa_ref[...], preferred_element_type=f32)
         + jnp.dot(kp_ref[...], w3a_ref[...], preferred_element_type=f32))
    b = (jnp.dot(kx_ref[...], wb_ref[...], preferred_element_type=f32)
         + jnp.dot(kp_ref[...], w3b_ref[...], preferred_element_type=f32))
    z_ref[...] = lax.shift_right_logical(_encode_top16(a), 16) | _encode_top16(b)
    c_ref[...] = (jnp.dot(qx_ref[...], wi_ref[...], preferred_element_type=f32)
                  - jnp.dot(qp_ref[...], w3_ref[...], preferred_element_type=f32)
                  + bias_ref[...])


def _tc_zc(kx, kp8, qx, qp8, wa, w3a, wb, w3b, wi, w3, bias2):
    grid = NPAD // TC_BLOCK
    full = lambda shape: pl.BlockSpec(shape, lambda i: (0,) * len(shape))
    return pl.pallas_call(
        _tc_body,
        grid=(grid,),
        in_specs=[
            pl.BlockSpec((TC_BLOCK, IN_DIM), lambda i: (i, 0)),
            pl.BlockSpec((TC_BLOCK, 8), lambda i: (i, 0)),
            pl.BlockSpec((TC_BLOCK, IN_DIM), lambda i: (i, 0)),
            pl.BlockSpec((TC_BLOCK, 8), lambda i: (i, 0)),
            full((IN_DIM, HALF)),
            full((8, HALF)),
            full((IN_DIM, HALF)),
            full((8, HALF)),
            full((IN_DIM, OUT_DIM)),
            full((8, OUT_DIM)),
            full((1, OUT_DIM)),
        ],
        out_specs=[
            pl.BlockSpec((TC_BLOCK, HALF), lambda i: (i, 0)),
            pl.BlockSpec((TC_BLOCK, OUT_DIM), lambda i: (i, 0)),
        ],
        out_shape=[
            jax.ShapeDtypeStruct((NPAD, HALF), jnp.int32),
            jax.ShapeDtypeStruct((NPAD, OUT_DIM), jnp.float32),
        ],
    )(kx, kp8, qx, qp8, wa, w3a, wb, w3b, wi, w3, bias2)


def _tc_epi_body(m_ref, c_ref, out_ref):
    m = m_ref[...]
    lo = _decode_top16(m << 16)
    hi = _decode_top16(m & jnp.int32(-65536))
    out_ref[...] = jnp.concatenate([lo, hi], axis=1) + c_ref[...]


def _tc_epilogue(m, c):
    grid = N // 2000
    return pl.pallas_call(
        _tc_epi_body,
        grid=(grid,),
        in_specs=[
            pl.BlockSpec((2000, HALF), lambda i: (i, 0)),
            pl.BlockSpec((2000, OUT_DIM), lambda i: (i, 0)),
        ],
        out_specs=pl.BlockSpec((2000, OUT_DIM), lambda i: (i, 0)),
        out_shape=jax.ShapeDtypeStruct((N, OUT_DIM), jnp.float32),
    )(m, c)


NCHUNKS = ROWS_PER_WORKER // CHUNK_ROWS


GRO = 128


def _sc_body(z_hbm, idx_hbm, out_hbm,
             ia0, ib0, ia1, ib1, g0, g1, ob, sem0, sem1):
    wid = lax.axis_index("c") * 16 + lax.axis_index("s")
    row0 = wid * ROWS_PER_WORKER

    nc = jnp.minimum(ROWS_PER_WORKER, N - row0) // CHUNK_ROWS

    def fire(t, ia, ib, gb, sem):
        fbase = (row0 + t * CHUNK_ROWS) * K
        pltpu.sync_copy(idx_hbm.at[pl.ds(fbase, GRO)], ia)
        pltpu.make_async_copy(z_hbm.at[ia], gb.at[pl.ds(0, GRO)], sem).start()
        pltpu.sync_copy(idx_hbm.at[pl.ds(fbase + GRO, GRO)], ib)
        pltpu.make_async_copy(z_hbm.at[ib], gb.at[pl.ds(GRO, GRO)], sem).start()

    def wait_compute(t, ia, ib, gb, sem):
        pltpu.make_async_copy(z_hbm.at[ia], gb.at[pl.ds(0, GRO)], sem).wait()
        pltpu.make_async_copy(z_hbm.at[ib], gb.at[pl.ds(GRO, GRO)], sem).wait()

        def row_body(g, carry):
            base = g * K
            for b in range(HALF // LANES):
                sl = pl.ds(b * LANES, LANES)
                v = gb[base, sl]
                acc_lo = v << 16
                acc_hi = v
                for k in range(1, K):
                    v = gb[base + k, sl]
                    acc_lo = jnp.maximum(acc_lo, v << 16)
                    acc_hi = jnp.maximum(acc_hi, v)
                ob[g, sl] = (lax.shift_right_logical(acc_lo, 16)
                             | (acc_hi & jnp.int32(-65536)))
            return carry

        lax.fori_loop(0, CHUNK_ROWS, row_body, 0)
        pltpu.sync_copy(ob, out_hbm.at[pl.ds(row0 + t * CHUNK_ROWS,
                                             CHUNK_ROWS)])

    fire(0, ia0, ib0, g0, sem0)

    def outer(j, carry):
        t0 = 2 * j
        fire(t0 + 1, ia1, ib1, g1, sem1)
        wait_compute(t0, ia0, ib0, g0, sem0)

        @pl.when(t0 + 2 < nc)
        def _():
            fire(t0 + 2, ia0, ib0, g0, sem0)

        wait_compute(t0 + 1, ia1, ib1, g1, sem1)
        return carry

    lax.fori_loop(0, nc // 2, outer, 0)

    @pl.when(nc % 2 == 1)
    def _():
        wait_compute(nc - 1, ia0, ib0, g0, sem0)


@functools.cache
def _sc_call():
    return pl.kernel(
        _sc_body,
        out_type=jax.ShapeDtypeStruct((N, HALF), jnp.int32),
        mesh=plsc.VectorSubcoreMesh(core_axis_name="c", subcore_axis_name="s"),
        scratch_types=[
            pltpu.VMEM((GRO,), jnp.int32),
            pltpu.VMEM((GRO,), jnp.int32),
            pltpu.VMEM((GRO,), jnp.int32),
            pltpu.VMEM((GRO,), jnp.int32),
            pltpu.VMEM((CHUNK_ROWS * K, HALF), jnp.int32),
            pltpu.VMEM((CHUNK_ROWS * K, HALF), jnp.int32),
            pltpu.VMEM((CHUNK_ROWS, HALF), jnp.int32),
            pltpu.SemaphoreType.DMA,
            pltpu.SemaphoreType.DMA,
        ],
    )


def kernel(query_pos, key_pos, idx_neighbors, query_x, key_x,
           W_xi, b_xi, W_xn, b_xn):
    kp8 = jnp.pad(key_pos, ((0, 0), (0, 5)))
    qp8 = jnp.pad(query_pos, ((0, 0), (0, 5)))
    w3 = jnp.pad(W_xn[:3], ((0, 5), (0, 0)))
    wx = W_xn[3:]
    wa, wb = wx[:, :HALF], wx[:, HALF:]
    w3a, w3b = w3[:, :HALF], w3[:, HALF:]
    bias2 = (b_xi + b_xn)[None, :]

    z, c = _tc_zc(key_x, kp8, query_x, qp8, wa, w3a, wb, w3b, W_xi, w3, bias2)

    idx_flat = idx_neighbors.astype(jnp.int32).reshape(-1)
    m = _sc_call()(z, idx_flat)
    return _tc_epilogue(m, c)
